# baseline (speedup 1.0000x reference)
import jax
import jax.numpy as jnp
from jax.experimental import pallas as pl
from jax.experimental.pallas import tpu as pltpu

_N = 4096
_CE = 4
_CN = 128
_COUT = 128

_BI = 256
_BK = 4096
_NI = _N // _BI
_NK = (_N * _CE) // _BK


def _body(a_ref, w2_ref, x_ref, wself_ref, b_ref, dinv_ref, o_ref, acc_ref):
    k = pl.program_id(1)

    @pl.when(k == 0)
    def _init():
        acc_ref[...] = jnp.zeros_like(acc_ref)

    acc_ref[...] += jnp.dot(
        a_ref[...], w2_ref[...], preferred_element_type=jnp.float32
    )

    @pl.when(k == _NK - 1)
    def _finish():
        msg = acc_ref[...] * dinv_ref[...]
        self_t = jnp.dot(
            x_ref[...], wself_ref[...], preferred_element_type=jnp.float32
        )
        o_ref[...] = jnp.maximum(msg + self_t + b_ref[...], 0.0)


def kernel(D, A, X, W_pass, b_pass, W_self, b_self):
    A2 = A.reshape(_N, _N * _CE)
    W2 = jnp.tile(W_pass.T, (_BK // _CE, 1))
    Wself_T = W_self.T
    b = (b_pass + b_self).reshape(1, _COUT)
    Dinv = (1.0 / D).reshape(_N, 1)

    out = pl.pallas_call(
        _body,
        grid=(_NI, _NK),
        in_specs=[
            pl.BlockSpec((_BI, _BK), lambda i, k: (i, k)),
            pl.BlockSpec((_BK, _COUT), lambda i, k: (0, 0)),
            pl.BlockSpec((_BI, _CN), lambda i, k: (i, 0)),
            pl.BlockSpec((_CN, _COUT), lambda i, k: (0, 0)),
            pl.BlockSpec((1, _COUT), lambda i, k: (0, 0)),
            pl.BlockSpec((_BI, 1), lambda i, k: (i, 0)),
        ],
        out_specs=pl.BlockSpec((_BI, _COUT), lambda i, k: (i, 0)),
        out_shape=jax.ShapeDtypeStruct((_N, _COUT), jnp.float32),
        scratch_shapes=[pltpu.VMEM((_BI, _COUT), jnp.float32)],
        compiler_params=pltpu.CompilerParams(
            dimension_semantics=("parallel", "arbitrary"),
        ),
    )(A2, W2, X, Wself_T, b, Dinv)
    return out


# speedup vs baseline: 1.0204x; 1.0204x over previous
import jax
import jax.numpy as jnp
from jax.experimental import pallas as pl
from jax.experimental.pallas import tpu as pltpu

_N = 4096
_CE = 4
_CN = 128
_COUT = 128

_BI = 256
_BK = 4096
_NI = _N // _BI
_NK = (_N * _CE) // _BK


def _body(a_ref, w2_ref, x_ref, wself_ref, b_ref, dinv_ref, o_ref, acc_ref):
    k = pl.program_id(1)

    @pl.when(k == 0)
    def _init():
        acc_ref[...] = jnp.zeros_like(acc_ref)

    a = a_ref[...]
    w = _BK
    while w > 128:
        w //= 2
        a = a[:, :w] + a[:, w:]
    acc_ref[...] += a

    @pl.when(k == _NK - 1)
    def _finish():
        msg = (
            jnp.dot(acc_ref[...], w2_ref[...], preferred_element_type=jnp.float32)
            * dinv_ref[...]
        )
        self_t = jnp.dot(
            x_ref[...], wself_ref[...], preferred_element_type=jnp.float32
        )
        o_ref[...] = jnp.maximum(msg + self_t + b_ref[...], 0.0)


def kernel(D, A, X, W_pass, b_pass, W_self, b_self):
    A2 = A.reshape(_N, _N * _CE)
    W2 = jnp.tile(W_pass.T, (128 // _CE, 1))
    Wself_T = W_self.T
    b = (b_pass + b_self).reshape(1, _COUT)
    Dinv = (1.0 / D).reshape(_N, 1)

    out = pl.pallas_call(
        _body,
        grid=(_NI, _NK),
        in_specs=[
            pl.BlockSpec((_BI, _BK), lambda i, k: (i, k)),
            pl.BlockSpec((128, _COUT), lambda i, k: (0, 0)),
            pl.BlockSpec((_BI, _CN), lambda i, k: (i, 0)),
            pl.BlockSpec((_CN, _COUT), lambda i, k: (0, 0)),
            pl.BlockSpec((1, _COUT), lambda i, k: (0, 0)),
            pl.BlockSpec((_BI, 1), lambda i, k: (i, 0)),
        ],
        out_specs=pl.BlockSpec((_BI, _COUT), lambda i, k: (i, 0)),
        out_shape=jax.ShapeDtypeStruct((_N, _COUT), jnp.float32),
        scratch_shapes=[pltpu.VMEM((_BI, _COUT), jnp.float32)],
        compiler_params=pltpu.CompilerParams(
            dimension_semantics=("parallel", "arbitrary"),
        ),
    )(A2, W2, X, Wself_T, b, Dinv)
    return out
